# baseline (speedup 1.0000x reference)
import functools

import jax
import jax.numpy as jnp
from jax import lax
from jax.experimental import pallas as pl
from jax.experimental.pallas import tpu as pltpu
from jax.experimental.pallas import tpu_sc as plsc

N = 10000
D_IN = 128
D_HID = 64
E = 320000

NC = 2
NS = 16
K = 125
CHUNKS = 80
NPAD = 10240
STRIPE = NPAD // NS
HALF = STRIPE // 2
ZB = 80
CW = 16
NBUF = 4
ROUNDS = CHUNKS // NBUF
L = 16


def _newton_rsqrt(deg):
    bits = lax.bitcast_convert_type(deg, jnp.int32)
    seed = jnp.full((L,), 0x5F3759DF, jnp.int32) - (bits >> 1)
    y = lax.bitcast_convert_type(seed, jnp.float32)
    for _ in range(3):
        y = y * (1.5 - 0.5 * deg * y * y)
    return y


def _sc_count_body(dst_hbm, out_hbm, dst_v, ones_v, acc_sh, sem):
    c = lax.axis_index("c")
    s = lax.axis_index("s")
    base = s * STRIPE

    def _zero_row(i, carry):
        ones_v[i, :] = jnp.zeros((CW,), jnp.float32)
        return carry

    lax.fori_loop(0, K, _zero_row, 0)

    def _zero_stripe(i, carry):
        pltpu.sync_copy(ones_v.at[pl.ds(0, ZB)], acc_sh.at[pl.ds(base + i * ZB, ZB)])
        return carry

    lax.fori_loop(0, STRIPE // ZB, _zero_stripe, 0)

    def _fill_row(i, carry):
        ones_v[i, :] = jnp.ones((CW,), jnp.float32)
        return carry

    lax.fori_loop(0, K, _fill_row, 0)

    pltpu.sync_copy(dst_hbm.at[c, s], dst_v)
    plsc.subcore_barrier()

    def _scatter(j, carry):
        pltpu.async_copy(ones_v, acc_sh.at[dst_v.at[j]], sem, add=True)
        return carry

    lax.fori_loop(0, CHUNKS, _scatter, 0)

    def _drain(j, carry):
        pltpu.make_async_copy(ones_v, acc_sh.at[dst_v.at[0]], sem).wait()
        return carry

    lax.fori_loop(0, CHUNKS, _drain, 0)
    plsc.subcore_barrier()

    pltpu.sync_copy(acc_sh.at[pl.ds(base, STRIPE)], out_hbm.at[c, pl.ds(base, STRIPE)])


def _sc_aggregate_body(
    src_hbm, dst_hbm, h_hbm, cnt_hbm, agg_out, g_out,
    src_v, dst_v, cnt_t, hv, *rest,
):
    rows = rest[:NBUF]
    acc_sh = rest[NBUF]
    sem_g = rest[NBUF + 1 :]
    c = lax.axis_index("c")
    s = lax.axis_index("s")
    base = s * STRIPE

    def _zero_rows0(i, carry):
        for kk in range(D_HID // L):
            rows[0][i, pl.ds(kk * L, L)] = jnp.zeros((L,), jnp.float32)
        return carry

    lax.fori_loop(0, K, _zero_rows0, 0)

    @pl.when(c == 1)
    def _():
        def _zero_acc(i, carry):
            pltpu.sync_copy(
                rows[0].at[pl.ds(0, ZB)], acc_sh.at[pl.ds(base + i * ZB, ZB)]
            )
            return carry

        lax.fori_loop(0, STRIPE // ZB, _zero_acc, 0)

    pltpu.sync_copy(src_hbm.at[c, s], src_v)
    pltpu.sync_copy(dst_hbm.at[c, s], dst_v)

    for hh in range(2):
        hbase = base + hh * HALF
        pltpu.sync_copy(h_hbm.at[pl.ds(hbase, HALF)], hv)
        pltpu.sync_copy(cnt_hbm.at[0, pl.ds(hbase, HALF)], cnt_t.at[0])
        pltpu.sync_copy(cnt_hbm.at[1, pl.ds(hbase, HALF)], cnt_t.at[1])

        def _row(row, carry):
            cv = cnt_t[0, row, :] + cnt_t[1, row, :]
            y = _newton_rsqrt(cv + 1.0)
            for kk in range(D_HID // L):
                sl = pl.ds(kk * L, L)
                hv[row, sl] = hv[row, sl] * y
            return carry

        lax.fori_loop(0, HALF, _row, 0)

        pltpu.sync_copy(hv, g_out.at[pl.ds(hbase, HALF)])

        @pl.when(c == 0)
        def _():
            pltpu.sync_copy(hv, acc_sh.at[pl.ds(hbase, HALF)])

    plsc.subcore_barrier()

    for b in range(NBUF):
        pltpu.async_copy(g_out.at[src_v.at[b]], rows[b], sem_g[b])

    def _round(r, carry):
        j0 = r * NBUF
        for b in range(NBUF):
            pltpu.make_async_copy(g_out.at[src_v.at[j0 + b]], rows[b], sem_g[b]).wait()
            pltpu.sync_copy(rows[b], acc_sh.at[dst_v.at[j0 + b]], add=True)

            @pl.when(r < ROUNDS - 1)
            def _():
                pltpu.async_copy(g_out.at[src_v.at[j0 + NBUF + b]], rows[b], sem_g[b])

        return carry

    lax.fori_loop(0, ROUNDS, _round, 0)
    plsc.subcore_barrier()

    pltpu.sync_copy(acc_sh.at[pl.ds(base, STRIPE)], agg_out.at[c, pl.ds(base, STRIPE)])


@functools.lru_cache(maxsize=None)
def _sc_kernels():
    mesh = plsc.VectorSubcoreMesh(
        core_axis_name="c", subcore_axis_name="s", num_cores=NC, num_subcores=NS
    )
    count = pl.kernel(
        _sc_count_body,
        out_type=jax.ShapeDtypeStruct((NC, NPAD, CW), jnp.float32),
        mesh=mesh,
        scratch_types=[
            pltpu.VMEM((CHUNKS, K), jnp.int32),
            pltpu.VMEM((K, CW), jnp.float32),
            pltpu.VMEM_SHARED((NPAD, CW), jnp.float32),
            pltpu.SemaphoreType.DMA,
        ],
        name="sc_degree_histogram",
    )
    aggregate = pl.kernel(
        _sc_aggregate_body,
        out_type=(
            jax.ShapeDtypeStruct((NC, NPAD, D_HID), jnp.float32),
            jax.ShapeDtypeStruct((NPAD, D_HID), jnp.float32),
        ),
        mesh=mesh,
        scratch_types=[
            pltpu.VMEM((CHUNKS, K), jnp.int32),
            pltpu.VMEM((CHUNKS, K), jnp.int32),
            pltpu.VMEM((NC, HALF, CW), jnp.float32),
            pltpu.VMEM((HALF, D_HID), jnp.float32),
        ]
        + [pltpu.VMEM((K, D_HID), jnp.float32) for _ in range(NBUF)]
        + [pltpu.VMEM_SHARED((NPAD, D_HID), jnp.float32)]
        + [pltpu.SemaphoreType.DMA for _ in range(NBUF)],
        name="sc_scale_aggregate",
        compiler_params=pltpu.CompilerParams(use_tc_tiling_on_sc=False),
    )
    return count, aggregate


def _h_body(x_ref, w_ref, h_ref):
    h = jnp.dot(x_ref[...], w_ref[...], preferred_element_type=jnp.float32)
    h_ref[0:N, :] = h
    h_ref[N:NPAD, :] = jnp.zeros((NPAD - N, D_HID), jnp.float32)


_h_call = pl.pallas_call(
    _h_body,
    out_shape=jax.ShapeDtypeStruct((NPAD, D_HID), jnp.float32),
)


def _dec_body(a0_ref, a1_ref, c0_ref, c1_ref, be_ref, wd_ref, bd_ref, z_ref, xr_ref):
    deg = c0_ref[:, 0:1] + c1_ref[:, 0:1] + 1.0
    dis = lax.rsqrt(deg)
    pre = (a0_ref[...] + a1_ref[...]) * dis + be_ref[...]
    z = jnp.maximum(pre, 0.0)
    z_ref[...] = z
    xr_ref[...] = (
        jnp.dot(z, wd_ref[...], preferred_element_type=jnp.float32) + bd_ref[...]
    )


_dec_call = pl.pallas_call(
    _dec_body,
    out_shape=(
        jax.ShapeDtypeStruct((N, D_HID), jnp.float32),
        jax.ShapeDtypeStruct((N, D_IN), jnp.float32),
    ),
)


def kernel(x, edge_index, W_enc, b_enc, W_dec, b_dec):
    src_p = edge_index[0].astype(jnp.int32).reshape(NC, NS, CHUNKS, K)
    dst_p = edge_index[1].astype(jnp.int32).reshape(NC, NS, CHUNKS, K)

    sc_count, sc_aggregate = _sc_kernels()
    h = _h_call(x, W_enc)
    counts = sc_count(dst_p)
    agg, _ = sc_aggregate(src_p, dst_p, h, counts)
    z, xr = _dec_call(
        agg[0, :N],
        agg[1, :N],
        counts[0, :N],
        counts[1, :N],
        b_enc.reshape(1, D_HID),
        W_dec,
        b_dec.reshape(1, D_IN),
    )
    return (z, xr)


# speedup vs baseline: 1.0176x; 1.0176x over previous
import functools

import jax
import jax.numpy as jnp
from jax import lax
from jax.experimental import pallas as pl
from jax.experimental.pallas import tpu as pltpu
from jax.experimental.pallas import tpu_sc as plsc

N = 10000
D_IN = 128
D_HID = 64
E = 320000

NC = 2
NS = 16
K = 125
CHUNKS = 80
NPAD = 10240
STRIPE = NPAD // NS
HALF = STRIPE // 2
ZB = 80
CW = 16
NBUF = 4
ROUNDS = CHUNKS // NBUF
L = 16


def _newton_rsqrt(deg):
    bits = lax.bitcast_convert_type(deg, jnp.int32)
    seed = jnp.full((L,), 0x5F3759DF, jnp.int32) - (bits >> 1)
    y = lax.bitcast_convert_type(seed, jnp.float32)
    for _ in range(3):
        y = y * (1.5 - 0.5 * deg * y * y)
    return y


def _sc_count_body(dst_hbm, out_hbm, dst_v, ones_v, acc_sh, sem):
    c = lax.axis_index("c")
    s = lax.axis_index("s")
    base = s * STRIPE

    def _zero_row(i, carry):
        ones_v[i, :] = jnp.zeros((CW,), jnp.float32)
        return carry

    lax.fori_loop(0, K, _zero_row, 0)

    def _zero_stripe(i, carry):
        pltpu.sync_copy(ones_v.at[pl.ds(0, ZB)], acc_sh.at[pl.ds(base + i * ZB, ZB)])
        return carry

    lax.fori_loop(0, STRIPE // ZB, _zero_stripe, 0)

    def _fill_row(i, carry):
        ones_v[i, :] = jnp.ones((CW,), jnp.float32)
        return carry

    lax.fori_loop(0, K, _fill_row, 0)

    pltpu.sync_copy(dst_hbm.at[c, s], dst_v)
    plsc.subcore_barrier()

    def _scatter(j, carry):
        pltpu.async_copy(ones_v, acc_sh.at[dst_v.at[j]], sem, add=True)
        return carry

    lax.fori_loop(0, CHUNKS, _scatter, 0)

    def _drain(j, carry):
        pltpu.make_async_copy(ones_v, acc_sh.at[dst_v.at[0]], sem).wait()
        return carry

    lax.fori_loop(0, CHUNKS, _drain, 0)
    plsc.subcore_barrier()

    pltpu.sync_copy(acc_sh.at[pl.ds(base, STRIPE)], out_hbm.at[c, pl.ds(base, STRIPE)])


def _sc_aggregate_body(
    src_hbm, dst_hbm, h_hbm, cnt_hbm, agg_out, g_out,
    src_v, dst_v, cnt_t, hv, *rest,
):
    rows = rest[:NBUF]
    acc_sh = rest[NBUF]
    sem_g = rest[NBUF + 1 : 2 * NBUF + 1]
    sem_ld = rest[2 * NBUF + 1 :]
    c = lax.axis_index("c")
    s = lax.axis_index("s")
    base = s * STRIPE

    def _zero_rows0(i, carry):
        for kk in range(D_HID // L):
            rows[0][i, pl.ds(kk * L, L)] = jnp.zeros((L,), jnp.float32)
        return carry

    lax.fori_loop(0, K, _zero_rows0, 0)

    @pl.when(c == 1)
    def _():
        def _zero_acc(i, carry):
            pltpu.sync_copy(
                rows[0].at[pl.ds(0, ZB)], acc_sh.at[pl.ds(base + i * ZB, ZB)]
            )
            return carry

        lax.fori_loop(0, STRIPE // ZB, _zero_acc, 0)

    def _fire_loads(hbase):
        pltpu.async_copy(h_hbm.at[pl.ds(hbase, HALF)], hv, sem_ld[0])
        pltpu.async_copy(cnt_hbm.at[0, pl.ds(hbase, HALF)], cnt_t.at[0], sem_ld[1])
        pltpu.async_copy(cnt_hbm.at[1, pl.ds(hbase, HALF)], cnt_t.at[1], sem_ld[2])

    def _wait_loads(hbase):
        pltpu.make_async_copy(h_hbm.at[pl.ds(hbase, HALF)], hv, sem_ld[0]).wait()
        pltpu.make_async_copy(
            cnt_hbm.at[0, pl.ds(hbase, HALF)], cnt_t.at[0], sem_ld[1]
        ).wait()
        pltpu.make_async_copy(
            cnt_hbm.at[1, pl.ds(hbase, HALF)], cnt_t.at[1], sem_ld[2]
        ).wait()

    _fire_loads(base)
    pltpu.sync_copy(src_hbm.at[c, s], src_v)
    pltpu.sync_copy(dst_hbm.at[c, s], dst_v)

    for hh in range(2):
        hbase = base + hh * HALF
        _wait_loads(hbase)

        def _row2(i, carry):
            for u in range(2):
                row = 2 * i + u
                cv = cnt_t[0, row, :] + cnt_t[1, row, :]
                y = _newton_rsqrt(cv + 1.0)
                for kk in range(D_HID // L):
                    sl = pl.ds(kk * L, L)
                    hv[row, sl] = hv[row, sl] * y
            return carry

        lax.fori_loop(0, HALF // 2, _row2, 0)

        pltpu.sync_copy(hv, g_out.at[pl.ds(hbase, HALF)])

        @pl.when(c == 0)
        def _():
            pltpu.sync_copy(hv, acc_sh.at[pl.ds(hbase, HALF)])

        if hh == 0:
            _fire_loads(base + HALF)

    plsc.subcore_barrier()

    for b in range(NBUF):
        pltpu.async_copy(g_out.at[src_v.at[b]], rows[b], sem_g[b])

    def _round(r, carry):
        j0 = r * NBUF
        for b in range(NBUF):
            pltpu.make_async_copy(g_out.at[src_v.at[j0 + b]], rows[b], sem_g[b]).wait()
            pltpu.sync_copy(rows[b], acc_sh.at[dst_v.at[j0 + b]], add=True)

            @pl.when(r < ROUNDS - 1)
            def _():
                pltpu.async_copy(g_out.at[src_v.at[j0 + NBUF + b]], rows[b], sem_g[b])

        return carry

    lax.fori_loop(0, ROUNDS, _round, 0)
    plsc.subcore_barrier()

    pltpu.sync_copy(acc_sh.at[pl.ds(base, STRIPE)], agg_out.at[c, pl.ds(base, STRIPE)])


@functools.lru_cache(maxsize=None)
def _sc_kernels():
    mesh = plsc.VectorSubcoreMesh(
        core_axis_name="c", subcore_axis_name="s", num_cores=NC, num_subcores=NS
    )
    count = pl.kernel(
        _sc_count_body,
        out_type=jax.ShapeDtypeStruct((NC, NPAD, CW), jnp.float32),
        mesh=mesh,
        scratch_types=[
            pltpu.VMEM((CHUNKS, K), jnp.int32),
            pltpu.VMEM((K, CW), jnp.float32),
            pltpu.VMEM_SHARED((NPAD, CW), jnp.float32),
            pltpu.SemaphoreType.DMA,
        ],
        name="sc_degree_histogram",
    )
    aggregate = pl.kernel(
        _sc_aggregate_body,
        out_type=(
            jax.ShapeDtypeStruct((NC, NPAD, D_HID), jnp.float32),
            jax.ShapeDtypeStruct((NPAD, D_HID), jnp.float32),
        ),
        mesh=mesh,
        scratch_types=[
            pltpu.VMEM((CHUNKS, K), jnp.int32),
            pltpu.VMEM((CHUNKS, K), jnp.int32),
            pltpu.VMEM((NC, HALF, CW), jnp.float32),
            pltpu.VMEM((HALF, D_HID), jnp.float32),
        ]
        + [pltpu.VMEM((K, D_HID), jnp.float32) for _ in range(NBUF)]
        + [pltpu.VMEM_SHARED((NPAD, D_HID), jnp.float32)]
        + [pltpu.SemaphoreType.DMA for _ in range(NBUF + 3)],
        name="sc_scale_aggregate",
        compiler_params=pltpu.CompilerParams(use_tc_tiling_on_sc=False),
    )
    return count, aggregate


def _h_body(x_ref, w_ref, h_ref):
    h = jnp.dot(x_ref[...], w_ref[...], preferred_element_type=jnp.float32)
    h_ref[0:N, :] = h
    h_ref[N:NPAD, :] = jnp.zeros((NPAD - N, D_HID), jnp.float32)


_h_call = pl.pallas_call(
    _h_body,
    out_shape=jax.ShapeDtypeStruct((NPAD, D_HID), jnp.float32),
)


def _dec_body(a0_ref, a1_ref, c0_ref, c1_ref, be_ref, wd_ref, bd_ref, z_ref, xr_ref):
    deg = c0_ref[:, 0:1] + c1_ref[:, 0:1] + 1.0
    dis = lax.rsqrt(deg)
    pre = (a0_ref[...] + a1_ref[...]) * dis + be_ref[...]
    z = jnp.maximum(pre, 0.0)
    z_ref[...] = z
    xr_ref[...] = (
        jnp.dot(z, wd_ref[...], preferred_element_type=jnp.float32) + bd_ref[...]
    )


_dec_call = pl.pallas_call(
    _dec_body,
    out_shape=(
        jax.ShapeDtypeStruct((N, D_HID), jnp.float32),
        jax.ShapeDtypeStruct((N, D_IN), jnp.float32),
    ),
)


def kernel(x, edge_index, W_enc, b_enc, W_dec, b_dec):
    src_p = edge_index[0].astype(jnp.int32).reshape(NC, NS, CHUNKS, K)
    dst_p = edge_index[1].astype(jnp.int32).reshape(NC, NS, CHUNKS, K)

    sc_count, sc_aggregate = _sc_kernels()
    h = _h_call(x, W_enc)
    counts = sc_count(dst_p)
    agg, _ = sc_aggregate(src_p, dst_p, h, counts)
    z, xr = _dec_call(
        agg[0, :N],
        agg[1, :N],
        counts[0, :N],
        counts[1, :N],
        b_enc.reshape(1, D_HID),
        W_dec,
        b_dec.reshape(1, D_IN),
    )
    return (z, xr)
